# Initial kernel scaffold; baseline (speedup 1.0000x reference)
import jax
import jax.numpy as jnp
from jax.experimental import pallas as pl


def kernel(node_attr, pos, edge_attr, edge_index, Wq0, Wk0, Wv0, We1_0, be1_0, We2_0, Wskip0, bskip0, Wq1, Wk1, Wv1, We1_1, be1_1, We2_1, Wskip1, bskip1, Wout, bout):
    raise NotImplementedError("write your pallas kernel here")


# speedup vs baseline: 32.2822x; 32.2822x over previous
import functools
import math

import jax
import jax.numpy as jnp
from jax import lax
from jax.experimental import pallas as pl
from jax.experimental.pallas import tpu as pltpu
from jax.experimental.pallas import tpu_sc as plsc

F32 = jnp.float32

_GRP = 128
_NW = 32
_BN = 2000
_BE = 3200
_R = 36


def _row_spec(b, d):
    return pl.BlockSpec((b, d), lambda i: (i, 0))


def _full_spec(shape):
    return pl.BlockSpec(shape, lambda i: (0,) * len(shape))


def _proj0_body(na, pos, wq, wk, wv, wsk, bsk, kv_o, q_o, sk_o):
    x = na[...]
    p = pos[...]
    z5 = jnp.zeros((x.shape[0], 5), F32)
    xk = jnp.dot(x, wk[...], preferred_element_type=F32)
    xv = jnp.dot(x, wv[...], preferred_element_type=F32)
    xq = jnp.dot(x, wq[...], preferred_element_type=F32)
    kv_o[...] = jnp.concatenate([xk, xv, p, z5], axis=1)
    q_o[...] = jnp.concatenate([xq, p, z5], axis=1)
    sk_o[...] = jnp.dot(x, wsk[...], preferred_element_type=F32) + bsk[...]


def _proj0(node_attr, pos, wq, wk, wv, wsk, bsk):
    n, d = node_attr.shape
    grid = (n // _BN,)
    return pl.pallas_call(
        _proj0_body,
        grid=grid,
        in_specs=[
            _row_spec(_BN, d), _row_spec(_BN, 3),
            _full_spec(wq.shape), _full_spec(wk.shape), _full_spec(wv.shape),
            _full_spec(wsk.shape), _full_spec(bsk.shape),
        ],
        out_specs=[_row_spec(_BN, 72), _row_spec(_BN, 40), _row_spec(_BN, 32)],
        out_shape=[
            jax.ShapeDtypeStruct((n, 72), F32),
            jax.ShapeDtypeStruct((n, 40), F32),
            jax.ShapeDtypeStruct((n, 32), F32),
        ],
    )(node_attr, pos, wq, wk, wv, wsk, bsk)


def _edge0_body(gs, gd, ea, we1, be1, we2, sel, selt, rows_o, rad_o):
    s = gs[...]
    d = gd[...]
    b = s.shape[0]
    xk = s[:, 0:32]
    xv = s[:, 32:64]
    ps = s[:, 64:67]
    xq = d[:, 0:32]
    pd = d[:, 32:35]
    rel = ps - pd
    dist = jnp.sqrt(jnp.sum(rel * rel, axis=1, keepdims=True) + 1e-8)
    rad = jnp.concatenate([ea[...], dist, jnp.zeros((b, 2), F32)], axis=1)
    rad_o[...] = rad
    hid = jax.nn.relu(jnp.dot(rad, we1[...], preferred_element_type=F32) + be1[...])
    emb = jnp.dot(hid, we2[...], preferred_element_type=F32)
    k = xk * emb
    v = xv * emb
    logits = jnp.dot(xq * k, sel[...], preferred_element_type=F32) * (1.0 / math.sqrt(8.0))
    w = jnp.exp(logits)
    wb = jnp.dot(w, selt[...], preferred_element_type=F32)
    rows_o[...] = jnp.concatenate([wb * v, w], axis=1)


def _edge0(gs, gd, ea, we1, be1, we2, sel, selt):
    e = gs.shape[0]
    grid = (e // _BE,)
    return pl.pallas_call(
        _edge0_body,
        grid=grid,
        in_specs=[
            _row_spec(_BE, gs.shape[1]), _row_spec(_BE, gd.shape[1]),
            _row_spec(_BE, 5),
            _full_spec(we1.shape), _full_spec(be1.shape), _full_spec(we2.shape),
            _full_spec(sel.shape), _full_spec(selt.shape),
        ],
        out_specs=[_row_spec(_BE, _R), _row_spec(_BE, 8)],
        out_shape=[
            jax.ShapeDtypeStruct((e, _R), F32),
            jax.ShapeDtypeStruct((e, 8), F32),
        ],
    )(gs, gd, ea, we1, be1, we2, sel, selt)


def _edge1_body(gs, gd, rad, we1, be1, we2, sel, selt, rows_o):
    s = gs[...]
    d = gd[...]
    b = s.shape[0]
    xk = s[:, 0:32]
    xv = s[:, 32:64]
    xq = d[...]
    hid = jax.nn.relu(jnp.dot(rad[...], we1[...], preferred_element_type=F32) + be1[...])
    emb = jnp.dot(hid, we2[...], preferred_element_type=F32)
    k = xk * emb
    v = xv * emb
    logits = jnp.dot(xq * k, sel[...], preferred_element_type=F32) * (1.0 / math.sqrt(8.0))
    w = jnp.exp(logits)
    wb = jnp.dot(w, selt[...], preferred_element_type=F32)
    rows_o[...] = jnp.concatenate([wb * v, w], axis=1)


def _edge1(gs, gd, rad, we1, be1, we2, sel, selt):
    e = gs.shape[0]
    grid = (e // _BE,)
    return pl.pallas_call(
        _edge1_body,
        grid=grid,
        in_specs=[
            _row_spec(_BE, 64), _row_spec(_BE, 32), _row_spec(_BE, 8),
            _full_spec(we1.shape), _full_spec(be1.shape), _full_spec(we2.shape),
            _full_spec(sel.shape), _full_spec(selt.shape),
        ],
        out_specs=[_row_spec(_BE, _R)],
        out_shape=[jax.ShapeDtypeStruct((e, _R), F32)],
    )(gs, gd, rad, we1, be1, we2, sel, selt)[0]


def _fin0_body(p0, p1, sk, wq, wk, wv, wsk, bsk, selt, kv_o, q_o, sk_o):
    p = p0[...] + p1[...]
    num = p[:, 0:32]
    den = p[:, 32:36]
    denb = jnp.dot(den, selt[...], preferred_element_type=F32) + 1e-9
    h = num / denb + sk[...]
    xk = jnp.dot(h, wk[...], preferred_element_type=F32)
    xv = jnp.dot(h, wv[...], preferred_element_type=F32)
    kv_o[...] = jnp.concatenate([xk, xv], axis=1)
    q_o[...] = jnp.dot(h, wq[...], preferred_element_type=F32)
    sk_o[...] = jnp.dot(h, wsk[...], preferred_element_type=F32) + bsk[...]


def _fin0(p0, p1, sk, wq, wk, wv, wsk, bsk, selt):
    n = p0.shape[0]
    grid = (n // _BN,)
    return pl.pallas_call(
        _fin0_body,
        grid=grid,
        in_specs=[
            _row_spec(_BN, _R), _row_spec(_BN, _R), _row_spec(_BN, 32),
            _full_spec(wq.shape), _full_spec(wk.shape), _full_spec(wv.shape),
            _full_spec(wsk.shape), _full_spec(bsk.shape), _full_spec(selt.shape),
        ],
        out_specs=[_row_spec(_BN, 64), _row_spec(_BN, 32), _row_spec(_BN, 32)],
        out_shape=[
            jax.ShapeDtypeStruct((n, 64), F32),
            jax.ShapeDtypeStruct((n, 32), F32),
            jax.ShapeDtypeStruct((n, 32), F32),
        ],
    )(p0, p1, sk, wq, wk, wv, wsk, bsk, selt)


def _fin1_body(p0, p1, sk, wout, bout, selt, out_o):
    p = p0[...] + p1[...]
    num = p[:, 0:32]
    den = p[:, 32:36]
    denb = jnp.dot(den, selt[...], preferred_element_type=F32) + 1e-9
    h = num / denb + sk[...]
    out_o[...] = jnp.dot(h, wout[...], preferred_element_type=F32) + bout[...]


def _fin1(p0, p1, sk, wout, bout, selt):
    n = p0.shape[0]
    grid = (n // _BN,)
    return pl.pallas_call(
        _fin1_body,
        grid=grid,
        in_specs=[
            _row_spec(_BN, _R), _row_spec(_BN, _R), _row_spec(_BN, 32),
            _full_spec(wout.shape), _full_spec(bout.shape), _full_spec(selt.shape),
        ],
        out_specs=[_row_spec(_BN, 32)],
        out_shape=[jax.ShapeDtypeStruct((n, 32), F32)],
    )(p0, p1, sk, wout, bout, selt)[0]


def _sc_gather(src2d, dst2d, kv_tab, q_tab):
    g_total = src2d.shape[0]
    e = g_total * _GRP
    kvd = kv_tab.shape[1]
    qd = q_tab.shape[1]
    n_iter = (g_total + _NW - 1) // _NW
    mesh = plsc.VectorSubcoreMesh(core_axis_name="c", subcore_axis_name="s")

    @functools.partial(
        pl.kernel,
        mesh=mesh,
        compiler_params=pltpu.CompilerParams(use_tc_tiling_on_sc=False),
        out_type=(
            jax.ShapeDtypeStruct((e, kvd), F32),
            jax.ShapeDtypeStruct((e, qd), F32),
        ),
        scratch_types=[
            pltpu.VMEM((2, _GRP), jnp.int32),
            pltpu.VMEM((_GRP, kvd), F32),
            pltpu.VMEM((_GRP, qd), F32),
            pltpu.SemaphoreType.DMA,
            pltpu.SemaphoreType.DMA,
        ],
    )
    def k(src_h, dst_h, kv_h, q_h, gsrc_h, gdst_h, idx_v, bufs_v, bufd_v, sem1, sem2):
        cid = lax.axis_index("c")
        sid = lax.axis_index("s")
        w = sid * 2 + cid

        def body(i, carry):
            g = w + i * _NW

            @pl.when(g < g_total)
            def _():
                pltpu.sync_copy(src_h.at[g], idx_v.at[0])
                pltpu.sync_copy(dst_h.at[g], idx_v.at[1])
                cp1 = pltpu.async_copy(kv_h.at[idx_v.at[0]], bufs_v, sem1)
                cp2 = pltpu.async_copy(q_h.at[idx_v.at[1]], bufd_v, sem2)
                cp1.wait()
                cp2.wait()
                pltpu.sync_copy(bufs_v, gsrc_h.at[pl.ds(g * _GRP, _GRP)])
                pltpu.sync_copy(bufd_v, gdst_h.at[pl.ds(g * _GRP, _GRP)])

            return carry

        lax.fori_loop(0, n_iter, body, 0)

    return k(src2d, dst2d, kv_tab, q_tab)


def _sc_scatter(rows, dst2d, zeros_pad):
    g_total = dst2d.shape[0]
    n_iter = (g_total + _NW - 1) // _NW
    n_pad = zeros_pad.shape[0]
    rpt = n_pad // 16
    mesh = plsc.VectorSubcoreMesh(core_axis_name="c", subcore_axis_name="s")

    @functools.partial(
        pl.kernel,
        mesh=mesh,
        compiler_params=pltpu.CompilerParams(use_tc_tiling_on_sc=False),
        out_type=jax.ShapeDtypeStruct((2, n_pad, _R), F32),
        scratch_types=[
            pltpu.VMEM((1, _GRP), jnp.int32),
            pltpu.VMEM((_GRP, _R), F32),
            pltpu.VMEM_SHARED((n_pad, _R), F32),
            pltpu.SemaphoreType.DMA,
        ],
    )
    def k(rows_h, dst_h, zeros_h, out_h, idx_v, rbuf_v, accum, sem):
        cid = lax.axis_index("c")
        sid = lax.axis_index("s")
        w = sid * 2 + cid

        pltpu.sync_copy(
            zeros_h.at[pl.ds(sid * rpt, rpt)],
            accum.at[pl.ds(sid * rpt, rpt)],
        )
        plsc.subcore_barrier()

        def body(i, carry):
            g = w + i * _NW

            @pl.when(g < g_total)
            def _():
                pltpu.sync_copy(dst_h.at[g], idx_v.at[0])
                pltpu.sync_copy(rows_h.at[pl.ds(g * _GRP, _GRP)], rbuf_v)
                pltpu.sync_copy(rbuf_v, accum.at[idx_v.at[0]], add=True)

            return carry

        lax.fori_loop(0, n_iter, body, 0)
        plsc.subcore_barrier()
        pltpu.sync_copy(
            accum.at[pl.ds(sid * rpt, rpt)],
            out_h.at[cid, pl.ds(sid * rpt, rpt)],
        )

    return k(rows, dst2d, zeros_pad)


def kernel(node_attr, pos, edge_attr, edge_index, Wq0, Wk0, Wv0, We1_0, be1_0,
           We2_0, Wskip0, bskip0, Wq1, Wk1, Wv1, We1_1, be1_1, We2_1, Wskip1,
           bskip1, Wout, bout):
    n = node_attr.shape[0]
    e = edge_attr.shape[0]
    c = Wout.shape[0]

    src2d = edge_index[0].reshape(e // _GRP, _GRP)
    dst2d = edge_index[1].reshape(e // _GRP, _GRP)

    sel = jnp.repeat(jnp.eye(4, dtype=F32), 8, axis=0)
    selt = jnp.transpose(sel)
    we1p0 = jnp.zeros((8, c), F32).at[:6].set(We1_0)
    we1p1 = jnp.zeros((8, c), F32).at[:6].set(We1_1)
    be1_0r = be1_0.reshape(1, c)
    be1_1r = be1_1.reshape(1, c)
    bsk0 = bskip0.reshape(1, c)
    bsk1 = bskip1.reshape(1, c)
    boutr = bout.reshape(1, c)

    n_pad = ((n + 3135) // 3136) * 3136
    zeros_pad = jnp.zeros((n_pad, _R), F32)

    kv0, q0, sk0 = _proj0(node_attr, pos, Wq0, Wk0, Wv0, Wskip0, bsk0)
    gs0, gd0 = _sc_gather(src2d, dst2d, kv0, q0)
    rows0, rad = _edge0(gs0, gd0, edge_attr, we1p0, be1_0r, We2_0, sel, selt)
    part0 = _sc_scatter(rows0, dst2d, zeros_pad)

    kv1, q1, sk1 = _fin0(part0[0, :n], part0[1, :n], sk0, Wq1, Wk1, Wv1, Wskip1, bsk1, selt)
    gs1, gd1 = _sc_gather(src2d, dst2d, kv1, q1)
    rows1 = _edge1(gs1, gd1, rad, we1p1, be1_1r, We2_1, sel, selt)
    part1 = _sc_scatter(rows1, dst2d, zeros_pad)

    return _fin1(part1[0, :n], part1[1, :n], sk1, Wout, boutr, selt)


# speedup vs baseline: 41.5005x; 1.2856x over previous
import functools
import math

import jax
import jax.numpy as jnp
from jax import lax
from jax.experimental import pallas as pl
from jax.experimental.pallas import tpu as pltpu
from jax.experimental.pallas import tpu_sc as plsc

F32 = jnp.float32

_GRP = 128
_NW = 32
_BN4 = 784
_BE4 = 2000

_SC_PARAMS = pltpu.CompilerParams(use_tc_tiling_on_sc=False)


def _row_spec(b, d):
    return pl.BlockSpec((b, d), lambda i: (i, 0))


def _full_spec(shape):
    return pl.BlockSpec(shape, lambda i: (0,) * len(shape))


def _proj0_body(x4, wq, wk, wv, wsk, bsk, xk_o, xv_o, xq_o, sk_o):
    x = x4[...]
    xk_o[...] = jnp.dot(x, wk[...], preferred_element_type=F32)
    xv_o[...] = jnp.dot(x, wv[...], preferred_element_type=F32)
    xq_o[...] = jnp.dot(x, wq[...], preferred_element_type=F32)
    sk_o[...] = jnp.dot(x, wsk[...], preferred_element_type=F32) + bsk[...]


def _proj0(x4, wq, wk, wv, wsk, bsk):
    m = x4.shape[0]
    grid = (m // _BN4,)
    out = jax.ShapeDtypeStruct((m, 128), F32)
    return pl.pallas_call(
        _proj0_body,
        grid=grid,
        in_specs=[
            _row_spec(_BN4, x4.shape[1]),
            _full_spec(wq.shape), _full_spec(wk.shape), _full_spec(wv.shape),
            _full_spec(wsk.shape), _full_spec(bsk.shape),
        ],
        out_specs=[_row_spec(_BN4, 128)] * 4,
        out_shape=[out, out, out, out],
    )(x4, wq, wk, wv, wsk, bsk)


def _edge_common(xk, xv, xq, rad, we1bd, be1t, we2bd, selbd, seltbd):
    emb = jnp.dot(
        jax.nn.relu(jnp.dot(rad, we1bd, preferred_element_type=F32) + be1t),
        we2bd, preferred_element_type=F32)
    k = xk * emb
    v = xv * emb
    lg = jnp.dot(xq * k, selbd, preferred_element_type=F32) * (1.0 / math.sqrt(8.0))
    w = jnp.exp(lg)
    wb = jnp.dot(w, seltbd, preferred_element_type=F32)
    return wb * v, w


def _edge0_body(xk4, xv4, xq4, ps4, pd4, rb4, we1bd, be1t, we2bd, selbd,
                seltbd, smat, qtmat, num_o, w_o, dx_o):
    b = xk4.shape[0]
    rel = ps4[...] - pd4[...]
    d2 = jnp.dot(rel * rel, smat[...], preferred_element_type=F32)
    lane = lax.broadcasted_iota(jnp.int32, (1, 32), 1)
    m5 = (lane % 8) == 5
    dist = jnp.where(m5, jnp.sqrt(d2 + 1e-8), 0.0)
    rad = rb4[...] + dist
    num, w = _edge_common(xk4[...], xv4[...], xq4[...], rad, we1bd[...],
                          be1t[...], we2bd[...], selbd[...], seltbd[...])
    num_o[...] = num
    w_o[...] = w
    dx_o[...] = jnp.dot(dist, qtmat[...], preferred_element_type=F32)


def _edge0(xk4, xv4, xq4, ps4, pd4, rb4, we1bd, be1t, we2bd, selbd, seltbd,
           smat, qtmat):
    m = xk4.shape[0]
    grid = (m // _BE4,)
    return pl.pallas_call(
        _edge0_body,
        grid=grid,
        in_specs=[
            _row_spec(_BE4, 128), _row_spec(_BE4, 128), _row_spec(_BE4, 128),
            _row_spec(_BE4, 128), _row_spec(_BE4, 128), _row_spec(_BE4, 32),
            _full_spec(we1bd.shape), _full_spec(be1t.shape),
            _full_spec(we2bd.shape), _full_spec(selbd.shape),
            _full_spec(seltbd.shape), _full_spec(smat.shape),
            _full_spec(qtmat.shape),
        ],
        out_specs=[_row_spec(_BE4, 128), _row_spec(_BE4, 16),
                   _row_spec(_BE4, 128)],
        out_shape=[
            jax.ShapeDtypeStruct((m, 128), F32),
            jax.ShapeDtypeStruct((m, 16), F32),
            jax.ShapeDtypeStruct((m, 128), F32),
        ],
    )(xk4, xv4, xq4, ps4, pd4, rb4, we1bd, be1t, we2bd, selbd, seltbd,
      smat, qtmat)


def _edge1_body(xk4, xv4, xq4, dx4, rb4, we1bd, be1t, we2bd, selbd, seltbd,
                qmat, num_o, w_o):
    rad = rb4[...] + jnp.dot(dx4[...], qmat[...], preferred_element_type=F32)
    num, w = _edge_common(xk4[...], xv4[...], xq4[...], rad, we1bd[...],
                          be1t[...], we2bd[...], selbd[...], seltbd[...])
    num_o[...] = num
    w_o[...] = w


def _edge1(xk4, xv4, xq4, dx4, rb4, we1bd, be1t, we2bd, selbd, seltbd, qmat):
    m = xk4.shape[0]
    grid = (m // _BE4,)
    return pl.pallas_call(
        _edge1_body,
        grid=grid,
        in_specs=[
            _row_spec(_BE4, 128), _row_spec(_BE4, 128), _row_spec(_BE4, 128),
            _row_spec(_BE4, 128), _row_spec(_BE4, 32),
            _full_spec(we1bd.shape), _full_spec(be1t.shape),
            _full_spec(we2bd.shape), _full_spec(selbd.shape),
            _full_spec(seltbd.shape), _full_spec(qmat.shape),
        ],
        out_specs=[_row_spec(_BE4, 128), _row_spec(_BE4, 16)],
        out_shape=[
            jax.ShapeDtypeStruct((m, 128), F32),
            jax.ShapeDtypeStruct((m, 16), F32),
        ],
    )(xk4, xv4, xq4, dx4, rb4, we1bd, be1t, we2bd, selbd, seltbd, qmat)


def _fin_common(pn0, pn1, pd0, pd1, sk, seltbd):
    den = pd0[...] + pd1[...]
    denb = jnp.dot(den, seltbd[...], preferred_element_type=F32) + 1e-9
    return (pn0[...] + pn1[...]) / denb + sk[...]


def _fin0_body(pn0, pn1, pd0, pd1, sk, wq, wk, wv, wsk, bsk, seltbd,
               xk_o, xv_o, xq_o, sk_o):
    h = _fin_common(pn0, pn1, pd0, pd1, sk, seltbd)
    xk_o[...] = jnp.dot(h, wk[...], preferred_element_type=F32)
    xv_o[...] = jnp.dot(h, wv[...], preferred_element_type=F32)
    xq_o[...] = jnp.dot(h, wq[...], preferred_element_type=F32)
    sk_o[...] = jnp.dot(h, wsk[...], preferred_element_type=F32) + bsk[...]


def _fin0(pn0, pn1, pd0, pd1, sk, wq, wk, wv, wsk, bsk, seltbd):
    m = sk.shape[0]
    grid = (m // _BN4,)
    out = jax.ShapeDtypeStruct((m, 128), F32)
    return pl.pallas_call(
        _fin0_body,
        grid=grid,
        in_specs=[
            _row_spec(_BN4, 128), _row_spec(_BN4, 128),
            _row_spec(_BN4, 16), _row_spec(_BN4, 16),
            _row_spec(_BN4, 128),
            _full_spec(wq.shape), _full_spec(wk.shape), _full_spec(wv.shape),
            _full_spec(wsk.shape), _full_spec(bsk.shape),
            _full_spec(seltbd.shape),
        ],
        out_specs=[_row_spec(_BN4, 128)] * 4,
        out_shape=[out, out, out, out],
    )(pn0, pn1, pd0, pd1, sk, wq, wk, wv, wsk, bsk, seltbd)


def _fin1_body(pn0, pn1, pd0, pd1, sk, wout, bout, seltbd, out_o):
    h = _fin_common(pn0, pn1, pd0, pd1, sk, seltbd)
    out_o[...] = jnp.dot(h, wout[...], preferred_element_type=F32) + bout[...]


def _fin1(pn0, pn1, pd0, pd1, sk, wout, bout, seltbd):
    m = sk.shape[0]
    grid = (m // _BN4,)
    return pl.pallas_call(
        _fin1_body,
        grid=grid,
        in_specs=[
            _row_spec(_BN4, 128), _row_spec(_BN4, 128),
            _row_spec(_BN4, 16), _row_spec(_BN4, 16),
            _row_spec(_BN4, 128),
            _full_spec(wout.shape), _full_spec(bout.shape),
            _full_spec(seltbd.shape),
        ],
        out_specs=[_row_spec(_BN4, 128)],
        out_shape=[jax.ShapeDtypeStruct((m, 128), F32)],
    )(pn0, pn1, pd0, pd1, sk, wout, bout, seltbd)[0]


def _sc_gather(src2d, dst2d, xk_t, xv_t, xq_t, pos_t=None):
    g_total = src2d.shape[0]
    e = g_total * _GRP
    n_iter = (g_total + _NW - 1) // _NW
    with_pos = pos_t is not None
    n_out = 5 if with_pos else 3
    mesh = plsc.VectorSubcoreMesh(core_axis_name="c", subcore_axis_name="s")
    out = jax.ShapeDtypeStruct((e, 32), F32)

    @functools.partial(
        pl.kernel,
        mesh=mesh,
        compiler_params=_SC_PARAMS,
        out_type=tuple([out] * n_out),
        scratch_types=[
            pltpu.VMEM((2, _GRP), jnp.int32),
        ] + [pltpu.VMEM((_GRP, 32), F32)] * n_out + [
            pltpu.SemaphoreType.DMA,
            pltpu.SemaphoreType.DMA,
        ],
    )
    def k(*refs):
        if with_pos:
            src_h, dst_h, kt, vt, qt, pt = refs[:6]
            outs = refs[6:11]
            idx_v = refs[11]
            bufs = refs[12:17]
            semg, semw = refs[17], refs[18]
        else:
            src_h, dst_h, kt, vt, qt = refs[:5]
            outs = refs[5:8]
            idx_v = refs[8]
            bufs = refs[9:12]
            semg, semw = refs[12], refs[13]
        cid = lax.axis_index("c")
        sid = lax.axis_index("s")
        w = sid * 2 + cid

        def body(i, carry):
            g = w + i * _NW

            @pl.when(g < g_total)
            def _():
                pltpu.sync_copy(src_h.at[g], idx_v.at[0])
                pltpu.sync_copy(dst_h.at[g], idx_v.at[1])
                cps = [
                    pltpu.async_copy(kt.at[idx_v.at[0]], bufs[0], semg),
                    pltpu.async_copy(vt.at[idx_v.at[0]], bufs[1], semg),
                    pltpu.async_copy(qt.at[idx_v.at[1]], bufs[2], semg),
                ]
                if with_pos:
                    cps.append(pltpu.async_copy(pt.at[idx_v.at[0]], bufs[3], semg))
                    cps.append(pltpu.async_copy(pt.at[idx_v.at[1]], bufs[4], semg))
                for cp in cps:
                    cp.wait()
                wcps = [
                    pltpu.async_copy(bufs[j], outs[j].at[pl.ds(g * _GRP, _GRP)], semw)
                    for j in range(n_out)
                ]
                for cp in wcps:
                    cp.wait()

            return carry

        lax.fori_loop(0, n_iter, body, 0)

    if with_pos:
        return k(src2d, dst2d, xk_t, xv_t, xq_t, pos_t)
    return k(src2d, dst2d, xk_t, xv_t, xq_t)


def _sc_scatter(num_lin, w_lin, dst2d, zeros32, zeros4):
    g_total = dst2d.shape[0]
    n_iter = (g_total + _NW - 1) // _NW
    n_pad = zeros32.shape[0]
    rpt = n_pad // 16
    mesh = plsc.VectorSubcoreMesh(core_axis_name="c", subcore_axis_name="s")

    @functools.partial(
        pl.kernel,
        mesh=mesh,
        compiler_params=_SC_PARAMS,
        out_type=(
            jax.ShapeDtypeStruct((2, n_pad, 32), F32),
            jax.ShapeDtypeStruct((2, n_pad, 4), F32),
        ),
        scratch_types=[
            pltpu.VMEM((1, _GRP), jnp.int32),
            pltpu.VMEM((_GRP, 32), F32),
            pltpu.VMEM((_GRP, 4), F32),
            pltpu.VMEM_SHARED((n_pad, 32), F32),
            pltpu.VMEM_SHARED((n_pad, 4), F32),
            pltpu.SemaphoreType.DMA,
        ],
    )
    def k(num_h, w_h, dst_h, z32_h, z4_h, on_h, od_h, idx_v, rbn_v, rbd_v,
          accn, accd, sem):
        cid = lax.axis_index("c")
        sid = lax.axis_index("s")
        w = sid * 2 + cid

        pltpu.sync_copy(z32_h.at[pl.ds(sid * rpt, rpt)],
                        accn.at[pl.ds(sid * rpt, rpt)])
        pltpu.sync_copy(z4_h.at[pl.ds(sid * rpt, rpt)],
                        accd.at[pl.ds(sid * rpt, rpt)])
        plsc.subcore_barrier()

        def body(i, carry):
            g = w + i * _NW

            @pl.when(g < g_total)
            def _():
                pltpu.sync_copy(dst_h.at[g], idx_v.at[0])
                cp1 = pltpu.async_copy(num_h.at[pl.ds(g * _GRP, _GRP)], rbn_v, sem)
                cp2 = pltpu.async_copy(w_h.at[pl.ds(g * _GRP, _GRP)], rbd_v, sem)
                cp1.wait()
                cp2.wait()
                pltpu.sync_copy(rbn_v, accn.at[idx_v.at[0]], add=True)
                pltpu.sync_copy(rbd_v, accd.at[idx_v.at[0]], add=True)

            return carry

        lax.fori_loop(0, n_iter, body, 0)
        plsc.subcore_barrier()
        pltpu.sync_copy(accn.at[pl.ds(sid * rpt, rpt)],
                        on_h.at[cid, pl.ds(sid * rpt, rpt)])
        pltpu.sync_copy(accd.at[pl.ds(sid * rpt, rpt)],
                        od_h.at[cid, pl.ds(sid * rpt, rpt)])

    return k(num_lin, w_lin, dst2d, zeros32, zeros4)


def kernel(node_attr, pos, edge_attr, edge_index, Wq0, Wk0, Wv0, We1_0, be1_0,
           We2_0, Wskip0, bskip0, Wq1, Wk1, Wv1, We1_1, be1_1, We2_1, Wskip1,
           bskip1, Wout, bout):
    n = node_attr.shape[0]
    e = edge_attr.shape[0]
    c = Wout.shape[0]
    e4 = e // 4
    n4 = n // 4
    n_pad = ((n + 3135) // 3136) * 3136

    src2d = edge_index[0].reshape(e // _GRP, _GRP)
    dst2d = edge_index[1].reshape(e // _GRP, _GRP)

    eye4 = jnp.eye(4, dtype=F32)
    sel = jnp.repeat(eye4, 8, axis=0)
    selbd = jnp.kron(eye4, sel)
    seltbd = jnp.kron(eye4, sel.T)

    def bd(wmat):
        return jnp.kron(eye4, wmat)

    we1p0 = jnp.zeros((8, c), F32).at[:6].set(We1_0)
    we1p1 = jnp.zeros((8, c), F32).at[:6].set(We1_1)

    smat = jnp.zeros((128, 32), F32)
    qtmat = jnp.zeros((32, 128), F32)
    qmat = jnp.zeros((128, 32), F32)
    for j in range(4):
        for t in range(3):
            smat = smat.at[32 * j + t, 8 * j + 5].set(1.0)
        qtmat = qtmat.at[8 * j + 5, 32 * j].set(1.0)
        qmat = qmat.at[32 * j, 8 * j + 5].set(1.0)

    x4 = jnp.pad(node_attr, ((0, n_pad - n), (0, 1))).reshape(n_pad // 4, 64)
    pos32 = jnp.pad(pos, ((0, 0), (0, 29)))
    rb4 = jnp.pad(edge_attr, ((0, 0), (0, 3))).reshape(e4, 32)
    zeros32 = jnp.zeros((n_pad, 32), F32)
    zeros4 = jnp.zeros((n_pad, 4), F32)

    def tile4(b):
        return jnp.tile(b.reshape(1, c), (1, 4)).reshape(1, 4 * c)

    xk0p, xv0p, xq0p, sk0p = _proj0(
        x4,
        bd(jnp.pad(Wq0, ((0, 1), (0, 0)))),
        bd(jnp.pad(Wk0, ((0, 1), (0, 0)))),
        bd(jnp.pad(Wv0, ((0, 1), (0, 0)))),
        bd(jnp.pad(Wskip0, ((0, 1), (0, 0)))),
        tile4(bskip0),
    )
    gxk, gxv, gxq, gps, gpd = _sc_gather(
        src2d, dst2d,
        xk0p.reshape(n_pad, 32), xv0p.reshape(n_pad, 32),
        xq0p.reshape(n_pad, 32), pos32)
    num0, w0, dx = _edge0(
        gxk.reshape(e4, 128), gxv.reshape(e4, 128), gxq.reshape(e4, 128),
        gps.reshape(e4, 128), gpd.reshape(e4, 128), rb4,
        bd(we1p0), tile4(be1_0), bd(We2_0), selbd, seltbd, smat, qtmat)
    pn, pd_ = _sc_scatter(num0.reshape(e, 32), w0.reshape(e, 4), dst2d,
                          zeros32, zeros4)

    xk1p, xv1p, xq1p, sk1p = _fin0(
        pn[0].reshape(n_pad // 4, 128),
        pn[1].reshape(n_pad // 4, 128),
        pd_[0].reshape(n_pad // 4, 16),
        pd_[1].reshape(n_pad // 4, 16),
        sk0p,
        bd(Wq1), bd(Wk1), bd(Wv1), bd(Wskip1), tile4(bskip1), seltbd)
    gxk1, gxv1, gxq1 = _sc_gather(
        src2d, dst2d,
        xk1p.reshape(n_pad, 32), xv1p.reshape(n_pad, 32),
        xq1p.reshape(n_pad, 32))
    num1, w1 = _edge1(
        gxk1.reshape(e4, 128), gxv1.reshape(e4, 128), gxq1.reshape(e4, 128),
        dx, rb4,
        bd(we1p1), tile4(be1_1), bd(We2_1), selbd, seltbd, qmat)
    pn1, pd1 = _sc_scatter(num1.reshape(e, 32), w1.reshape(e, 4), dst2d,
                           zeros32, zeros4)

    out4 = _fin1(
        pn1[0].reshape(n_pad // 4, 128),
        pn1[1].reshape(n_pad // 4, 128),
        pd1[0].reshape(n_pad // 4, 16),
        pd1[1].reshape(n_pad // 4, 16),
        sk1p,
        bd(Wout), tile4(bout), seltbd)

    return out4.reshape(n_pad, 32)[:n]
